# baseline (speedup 1.0000x reference)
import functools

import jax
import jax.numpy as jnp
import numpy as np
from jax import lax
from jax.experimental import pallas as pl
from jax.experimental.pallas import tpu as pltpu
from jax.experimental.pallas import tpu_sc as plsc

_T, _D = 50, 10
_PER = 2 ** (_D + 1) - 1
_N = _T * _PER
_C, _M, _G = 8, 256, 4
_F = _C * _G

_TB = 25
_NB = _T // _TB
_LBLK = _PER * _TB
_LINT = (2 ** _D - 1) * _TB
_LLEAF = (2 ** _D) * _TB
_LPAD = ((_LINT + 255) // 256) * 256
_NTOT = _NB * _LPAD
_NW = 32
_BPW = _NTOT // _NW
_EMIS_CHUNK = 2048


def _fold(a, l):
    for _ in range(l):
        h = a.shape[1] // 2
        a = a[:, :h] + a[:, h:]
    return a


def _csum(a):
    n = a.shape[-1]
    return jnp.sum(a.reshape(_G, _C, n), axis=1)


def _fold_csum(a, l):
    return _csum(_fold(a, l))


def _mm(m, a):
    return jax.lax.dot_general(
        m, a, dimension_numbers=(((1,), (0,)), ((), ())),
        preferred_element_type=jnp.float32)


_TW = 128


def _tab_body(b_ref, out_ref):
    bl = b_ref[...]
    bm = bl - jnp.max(bl, axis=0, keepdims=True)
    be = jnp.exp(bm)
    bs = jnp.sum(be, axis=0, keepdims=True)
    zeros = jnp.zeros((_M, _TW - 2 * _F), jnp.float32)
    out_ref[...] = jnp.concatenate(
        [be / bs, bm - jnp.log(bs), zeros], axis=1)


_CH = 400


def _sc_gather(tab, idx):
    mesh = plsc.VectorSubcoreMesh(core_axis_name="c", subcore_axis_name="s")
    n_chunks = _BPW // _CH

    @functools.partial(
        pl.kernel, mesh=mesh,
        out_type=jax.ShapeDtypeStruct((_NTOT, _TW), jnp.float32),
        scratch_types=[
            pltpu.VMEM((_BPW,), jnp.int32),
            pltpu.VMEM((_CH, _TW), jnp.float32),
            pltpu.VMEM((_CH, _TW), jnp.float32),
            pltpu.SemaphoreType.DMA,
            pltpu.SemaphoreType.DMA,
            pltpu.SemaphoreType.DMA,
        ],
    )
    def k(tab_hbm, idx_hbm, out_hbm, idx_v, rows0, rows1, gsem, wsem0, wsem1):
        wid = lax.axis_index("s") * 2 + lax.axis_index("c")
        base = wid * _BPW
        pltpu.sync_copy(idx_hbm.at[pl.ds(base, _BPW)], idx_v)
        bufs = (rows0, rows1)
        wsems = (wsem0, wsem1)
        writes = [None, None]
        for ci in range(n_chunks):
            b = ci % 2
            if writes[b] is not None:
                writes[b].wait()
            pltpu.async_copy(
                tab_hbm.at[idx_v.at[pl.ds(ci * _CH, _CH)]], bufs[b], gsem,
            ).wait()
            writes[b] = pltpu.async_copy(
                bufs[b], out_hbm.at[pl.ds(base + ci * _CH, _CH)], wsems[b])
        for w in writes:
            if w is not None:
                w.wait()

    return k(tab, idx)


def _bp_body(gx_ref, xl_ref, a_ref, att_ref, b_ref, pi_ref, out_ref):
    xl = xl_ref[0]
    at = a_ref[...]
    am = at - jnp.max(at, axis=1, keepdims=True)
    ae = jnp.exp(am)
    sm_a = ae / jnp.sum(ae, axis=1, keepdims=True)

    att = att_ref[...]
    atm = att - jnp.max(att, axis=2, keepdims=True)
    ate = jnp.exp(atm)
    sm_at = ate / jnp.sum(ate, axis=2, keepdims=True)
    al = sm_at * jnp.log(sm_at)

    pil = pi_ref[...].reshape(_G, _C, 1)
    pm = pil - jnp.max(pil, axis=1, keepdims=True)
    pe = jnp.exp(pm)
    ps = jnp.sum(pe, axis=1, keepdims=True)
    sm_pi = (pe / ps).reshape(_F, 1)
    log_pi = (pm - jnp.log(ps)).reshape(_F, 1)

    bl2 = b_ref[...]
    bm2 = bl2 - jnp.max(bl2, axis=1, keepdims=True)
    be2 = jnp.exp(bm2)
    bs2 = jnp.sum(be2, axis=1, keepdims=True)
    btab = be2 / bs2
    lbtab = bm2 - jnp.log(bs2)
    tabs16 = jnp.concatenate([btab, lbtab], axis=0).astype(jnp.bfloat16)

    colg = jax.lax.broadcasted_iota(jnp.int32, (_F, _F), 1) // _C
    rowg = jax.lax.broadcasted_iota(jnp.int32, (_F, _F), 0) // _C
    mask = (colg == rowg).astype(jnp.float32)
    bd_up = jnp.concatenate([sm_a] * _G, axis=2).reshape(_F, _F) * mask
    bd_dn = jnp.concatenate([sm_at] * _G, axis=2).reshape(_F, _F) * mask
    bd_al = jnp.concatenate([al] * _G, axis=2).reshape(_F, _F) * mask
    ones_bd = mask

    def emis(lo, n):
        rows = gx_ref[0, pl.ds(lo, n), :]
        t = jnp.transpose(rows)
        return t[:_F], t[_F:2 * _F]

    def emis_leaf():
        bx, lbx = [], []
        iota = jax.lax.broadcasted_iota(jnp.int32, (_M, 1), 0)
        for c0 in range(0, _LLEAF, _EMIS_CHUNK):
            cw = min(_EMIS_CHUNK, _LLEAF - c0)
            xs = xl[:, c0:c0 + cw]
            oh = (xs == iota).astype(jnp.bfloat16)
            r = _mm(tabs16, oh)
            bx.append(r[:_F])
            lbx.append(r[_F:])
        return (jnp.concatenate(bx, axis=1), jnp.concatenate(lbx, axis=1))

    def lanes(l):
        return (2 ** l) * _TB

    def lane_off(l):
        return (2 ** l - 1) * _TB

    beta = [None] * (_D + 1)
    tbeta = [None] * _D
    logb = [None] * (_D + 1)

    nl = lanes(_D)
    bx, logb[_D] = emis_leaf()
    b0 = sm_pi * bx
    beta[_D] = b0 / _mm(ones_bd, b0)

    for l in range(_D - 1, -1, -1):
        npa = lanes(l)
        bch = beta[l + 1]
        bmean = 0.5 * (bch[:, :npa] + bch[:, npa:])
        tb = _mm(bd_up, bmean)
        tbeta[l] = tb
        bx, logb[l] = emis(lane_off(l), npa)
        bb = tb * bx
        beta[l] = bb / _mm(ones_bd, bb)

    ll4 = jnp.zeros((_G, _TB), dtype=jnp.float32)
    eps = beta[0]
    for l in range(_D):
        npa = lanes(l)
        ll4 = ll4 + _fold_csum(eps * logb[l], l)
        w = eps / tbeta[l]
        s = _mm(bd_dn, w)
        cal = _mm(bd_al, w)
        b1 = beta[l + 1][:, :npa]
        b2 = beta[l + 1][:, npa:]
        ll4 = ll4 + _fold_csum(cal * (0.5 * (b1 + b2)), l)
        eps = jnp.concatenate([b1 * s, b2 * s], axis=1)

    ll4 = ll4 + _fold_csum(eps * logb[_D], _D)
    ll4 = ll4 + _fold_csum(eps * log_pi, _D)
    out_ref[0] = -ll4


def _relayout(x):
    xr = x.reshape(_T, _PER)
    pieces = []
    for l in range(_D + 1):
        seg = xr[:, 2 ** l - 1: 2 ** (l + 1) - 1]
        if l > 0:
            seg = seg.reshape((_T,) + (2,) * l)
            seg = jnp.transpose(seg, (0,) + tuple(range(l, 0, -1)))
            seg = seg.reshape(_T, 2 ** l)
        seg = seg.reshape(_NB, _TB, 2 ** l)
        seg = jnp.transpose(seg, (0, 2, 1))
        pieces.append(seg.reshape(_NB, (2 ** l) * _TB))
    return jnp.concatenate(pieces, axis=1)


@jax.jit
def _run(x, A, B, Pi):
    xp = _relayout(x)
    xint = jnp.pad(xp[:, :_LINT], ((0, 0), (0, _LPAD - _LINT)))
    xq = xint.reshape(-1)
    xleaf = xp[:, _LINT:].reshape(_NB, 1, _LLEAF)
    bn = jnp.transpose(B, (1, 2, 0)).reshape(_M, _F)
    tab = pl.pallas_call(
        _tab_body,
        out_shape=jax.ShapeDtypeStruct((_M, _TW), jnp.float32),
    )(bn)
    gx = _sc_gather(tab, xq)
    gx3 = gx.reshape(_NB, _LPAD, _TW)

    at = jnp.transpose(A, (2, 0, 1))
    att = jnp.transpose(A, (2, 1, 0))
    bt = jnp.transpose(B, (2, 0, 1)).reshape(_F, _M)
    pit = jnp.transpose(Pi, (1, 0)).reshape(_F, 1)
    out = pl.pallas_call(
        _bp_body,
        grid=(_NB,),
        in_specs=[
            pl.BlockSpec((1, _LPAD, _TW), lambda b: (b, 0, 0)),
            pl.BlockSpec((1, 1, _LLEAF), lambda b: (b, 0, 0)),
            pl.BlockSpec((_G, _C, _C), lambda b: (0, 0, 0)),
            pl.BlockSpec((_G, _C, _C), lambda b: (0, 0, 0)),
            pl.BlockSpec((_F, _M), lambda b: (0, 0)),
            pl.BlockSpec((_F, 1), lambda b: (0, 0)),
        ],
        out_specs=pl.BlockSpec((1, _G, _TB), lambda b: (b, 0, 0)),
        out_shape=jax.ShapeDtypeStruct((_NB, _G, _TB), jnp.float32),
    )(gx3, xleaf, at, att, bt, pit)
    return jnp.transpose(out, (0, 2, 1)).reshape(_T, _G)


def kernel(x, A, B, Pi, leaves, roots, inv_map, trees_ind, internal, levels):
    return _run(x, A, B, Pi)
